# baseline (speedup 1.0000x reference)
import functools

import jax
import jax.numpy as jnp
from jax import lax
from jax.experimental import pallas as pl
from jax.experimental.pallas import tpu as pltpu
from jax.experimental.pallas import tpu_sc as plsc

_NUM_CORES = 2
_NUM_SUBCORES = 16
_NUM_WORKERS = _NUM_CORES * _NUM_SUBCORES
_LANES = 16


@functools.lru_cache(maxsize=None)
def _build_lookup(batch: int, hist: int, dim: int, vocab: int):
    bw = batch // _NUM_WORKERS
    assert batch % (_NUM_WORKERS * _LANES) == 0
    n_groups = bw // _LANES

    mesh = plsc.VectorSubcoreMesh(core_axis_name="c", subcore_axis_name="s")

    @functools.partial(
        pl.kernel,
        mesh=mesh,
        out_type=jax.ShapeDtypeStruct((hist, dim, batch), jnp.float32),
        scratch_types=[
            pltpu.VMEM((bw,), jnp.int32),
            pltpu.VMEM((bw,), jnp.int32),
            pltpu.VMEM((bw, 2 * dim), jnp.float32),
            pltpu.VMEM((dim, bw), jnp.float32),
            pltpu.SemaphoreType.DMA,
        ],
        compiler_params=pltpu.CompilerParams(needs_layout_passes=False),
    )
    def lookup_kernel(tok_hbm, w2_hbm, out_hbm, idx_v, pp_v, pair_v, outb_v, sem):
        wid = lax.axis_index("s") * _NUM_CORES + lax.axis_index("c")
        b0 = wid * bw
        lane_iota = lax.iota(jnp.int32, _LANES)

        def do_h(h, carry):
            pltpu.sync_copy(tok_hbm.at[pl.ds(h * batch + b0, bw)], idx_v)
            for q in range(n_groups):
                t = idx_v[pl.ds(_LANES * q, _LANES)]
                pp_v[pl.ds(_LANES * q, _LANES)] = lax.shift_right_logical(t, 1)
            pltpu.async_copy(w2_hbm.at[pp_v], pair_v, sem).wait()
            for q in range(n_groups):
                t = idx_v[pl.ds(_LANES * q, _LANES)]
                col0 = lax.shift_left(lax.bitwise_and(t, 1), 6)
                j_vec = lane_iota + _LANES * q
                for c in range(dim):
                    vals = plsc.load_gather(pair_v, [j_vec, col0 + c])
                    outb_v[c, pl.ds(_LANES * q, _LANES)] = vals
            pltpu.sync_copy(outb_v, out_hbm.at[h, :, pl.ds(b0, bw)])
            return carry

        lax.fori_loop(0, hist, do_h, 0)

    return lookup_kernel


def kernel(token_ids, weight):
    batch, hist = token_ids.shape
    vocab, dim = weight.shape
    tok_flat = token_ids.T.reshape(-1).astype(jnp.int32)
    w2 = weight.reshape(vocab // 2, 2 * dim)
    out_t = _build_lookup(batch, hist, dim, vocab)(tok_flat, w2)
    return out_t.transpose(2, 0, 1)


# speedup vs baseline: 1.6810x; 1.6810x over previous
import functools

import jax
import jax.numpy as jnp
from jax import lax
from jax.experimental import pallas as pl
from jax.experimental.pallas import tpu as pltpu
from jax.experimental.pallas import tpu_sc as plsc

_NUM_CORES = 2
_NUM_SUBCORES = 16
_NUM_WORKERS = _NUM_CORES * _NUM_SUBCORES
_LANES = 16


@functools.lru_cache(maxsize=None)
def _build_lookup(batch: int, hist: int, dim: int, vocab: int):
    bw = batch // _NUM_WORKERS
    assert batch % (_NUM_WORKERS * _LANES) == 0
    n_groups = bw // _LANES

    mesh = plsc.VectorSubcoreMesh(core_axis_name="c", subcore_axis_name="s")

    @functools.partial(
        pl.kernel,
        mesh=mesh,
        out_type=jax.ShapeDtypeStruct((hist, dim, batch), jnp.float32),
        scratch_types=(
            [pltpu.VMEM((bw,), jnp.int32) for _ in range(2)]
            + [pltpu.VMEM((bw,), jnp.int32) for _ in range(2)]
            + [pltpu.VMEM((bw,), jnp.int32) for _ in range(2)]
            + [pltpu.VMEM((bw, 2 * dim), jnp.float32) for _ in range(2)]
            + [pltpu.VMEM((dim, bw), jnp.float32) for _ in range(2)]
            + [pltpu.SemaphoreType.DMA for _ in range(6)]
        ),
        compiler_params=pltpu.CompilerParams(needs_layout_passes=False),
    )
    def lookup_kernel(tok_hbm, w2_hbm, out_hbm, *scr):
        idx_v = scr[0:2]
        pp_v = scr[2:4]
        col_v = scr[4:6]
        pair_v = scr[6:8]
        outb_v = scr[8:10]
        sem_i = scr[10:12]
        sem_g = scr[12:14]
        sem_o = scr[14:16]

        wid = lax.axis_index("s") * _NUM_CORES + lax.axis_index("c")
        b0 = wid * bw
        lane_iota = lax.iota(jnp.int32, _LANES)

        def idx_start(h, b):
            pltpu.async_copy(tok_hbm.at[pl.ds(h * batch + b0, bw)], idx_v[b], sem_i[b])

        def idx_wait(h, b):
            pltpu.make_async_copy(
                tok_hbm.at[pl.ds(h * batch + b0, bw)], idx_v[b], sem_i[b]
            ).wait()

        def prep(b):
            for q in range(n_groups):
                sl = pl.ds(_LANES * q, _LANES)
                t = idx_v[b][sl]
                pp_v[b][sl] = lax.shift_right_logical(t, 1)
                col_v[b][sl] = lax.shift_left(lax.bitwise_and(t, 1), 6)

        def gather_start(b):
            pltpu.async_copy(w2_hbm.at[pp_v[b]], pair_v[b], sem_g[b])

        def gather_wait(b):
            pltpu.make_async_copy(w2_hbm.at[pp_v[b]], pair_v[b], sem_g[b]).wait()

        def transpose(b):
            for q in range(n_groups):
                sl = pl.ds(_LANES * q, _LANES)
                col0 = col_v[b][sl]
                j_vec = lane_iota + _LANES * q
                for cb in range(0, dim, 8):
                    vals = [
                        plsc.load_gather(pair_v[b], [j_vec, col0 + c])
                        for c in range(cb, cb + 8)
                    ]
                    for i, c in enumerate(range(cb, cb + 8)):
                        outb_v[b][c, sl] = vals[i]

        def store_start(h, b):
            pltpu.async_copy(outb_v[b], out_hbm.at[h, :, pl.ds(b0, bw)], sem_o[b])

        def store_wait(h, b):
            pltpu.make_async_copy(
                outb_v[b], out_hbm.at[h, :, pl.ds(b0, bw)], sem_o[b]
            ).wait()

        idx_start(0, 0)

        def do_group(t, carry):
            for b in range(2):
                h = 2 * t + b
                pb = 1 - b
                idx_wait(h, b)
                prep(b)
                gather_start(b)

                @pl.when(h + 1 < hist)
                def _():
                    idx_start(h + 1, pb)

                @pl.when(h >= 1)
                def _():
                    @pl.when(h >= 3)
                    def _():
                        store_wait(h - 3, pb)

                    gather_wait(pb)
                    transpose(pb)
                    store_start(h - 1, pb)

            return carry

        lax.fori_loop(0, hist // 2, do_group, 0)

        last = hist - 1
        lb = last % 2
        store_wait(last - 2, lb)
        gather_wait(lb)
        transpose(lb)
        store_start(last, lb)
        store_wait(last - 1, 1 - lb)
        store_wait(last, lb)

    return lookup_kernel


def kernel(token_ids, weight):
    batch, hist = token_ids.shape
    vocab, dim = weight.shape
    tok_flat = token_ids.T.reshape(-1).astype(jnp.int32)
    w2 = weight.reshape(vocab // 2, 2 * dim)
    out_t = _build_lookup(batch, hist, dim, vocab)(tok_flat, w2)
    return out_t.transpose(2, 0, 1)
